# baseline (speedup 1.0000x reference)
import jax
import jax.numpy as jnp
from jax import lax
from jax.experimental import pallas as pl
from jax.experimental.pallas import tpu as pltpu

_SEQ = 2048
_NOUT = 1000
_NIN = 1000
_TBLK = 512
_SUBL = _TBLK // 128


def _body(seq_ref, delta_ref, w_ref, b_ref, o_ref, srow, wbc, dscr, carry, wstat):
    i = pl.program_id(0)

    @pl.when(i == 0)
    def _():
        carry[0, 0] = jnp.float32(0.0)
        r0 = lax.broadcasted_iota(jnp.int32, (_NOUT, _NOUT), 0)
        c0 = lax.broadcasted_iota(jnp.int32, (_NOUT, _NOUT), 1)
        iden = (r0 == c0).astype(jnp.float32)
        tdims = (((1,), (1,)), ((), ()))
        wbc[:, 0:1] = lax.dot_general(
            iden, w_ref[...], tdims,
            preferred_element_type=jnp.float32,
            precision=lax.Precision.HIGHEST,
        )
        wbc[:, 1:2] = lax.dot_general(
            iden, b_ref[...], tdims,
            preferred_element_type=jnp.float32,
            precision=lax.Precision.HIGHEST,
        )
        dscr[0:1, 0:_NIN] = delta_ref[...]
        wstat[0, 0] = jnp.max(w_ref[...])
        wstat[0, 1] = jnp.min(w_ref[...])
        wstat[0, 2] = jnp.max(b_ref[...])

    for k in range(_SUBL):
        srow[0:1, k * 128 : (k + 1) * 128] = seq_ref[0, k : k + 1, :]

    dnums = lax.GatherDimensionNumbers(
        offset_dims=(),
        collapsed_slice_dims=(1,),
        start_index_map=(1,),
        operand_batching_dims=(0,),
        start_indices_batching_dims=(0,),
    )
    idx = srow[...]
    idxm = (idx & 127)[:, :, None]
    idxh = idx >> 7
    g = jnp.zeros((1, _TBLK), jnp.float32)
    for c in range(1024 // 128):
        gc = lax.gather(
            dscr[:, c * 128 : (c + 1) * 128],
            idxm,
            dimension_numbers=dnums,
            slice_sizes=(1, 1),
            mode=lax.GatherScatterMode.PROMISE_IN_BOUNDS,
        )
        g = jnp.where(idxh == c, gc, g)

    lane = lax.broadcasted_iota(jnp.int32, (1, _TBLK), 1)
    csum = g
    s = 1
    while s < _TBLK:
        rolled = pltpu.roll(csum, s, 1)
        csum = csum + jnp.where(lane >= s, rolled, jnp.float32(0.0))
        s *= 2
    c0 = carry[0, 0]
    counters = csum + c0
    carry[0, 0] = c0 + jnp.sum(g)

    mb = jnp.maximum(counters * wstat[0, 0], counters * wstat[0, 1]) + wstat[0, 2]
    e = jnp.exp(wbc[:, 0:1] * counters + (wbc[:, 1:2] - mb))
    s = lax.dot_general(
        jnp.ones((1, _NOUT), jnp.float32), e, (((1,), (0,)), ((), ())),
        preferred_element_type=jnp.float32,
    )
    o_ref[...] = e * (1.0 / s)


def kernel(input_seq, delta, W, b):
    seq3d = input_seq.reshape(_SEQ // _TBLK, _SUBL, 128)
    out_t = pl.pallas_call(
        _body,
        grid=(_SEQ // _TBLK,),
        in_specs=[
            pl.BlockSpec((1, _SUBL, 128), lambda i: (i, 0, 0)),
            pl.BlockSpec((1, _NIN), lambda i: (0, 0)),
            pl.BlockSpec((1, _NOUT), lambda i: (0, 0)),
            pl.BlockSpec((1, _NOUT), lambda i: (0, 0)),
        ],
        out_specs=pl.BlockSpec((_NOUT, _TBLK), lambda i: (0, i)),
        out_shape=jax.ShapeDtypeStruct((_NOUT, _SEQ), jnp.float32),
        scratch_shapes=[
            pltpu.VMEM((1, _TBLK), jnp.int32),
            pltpu.VMEM((_NOUT, 2), jnp.float32),
            pltpu.VMEM((1, 1024), jnp.float32),
            pltpu.SMEM((1, 1), jnp.float32),
            pltpu.SMEM((1, 3), jnp.float32),
        ],
    )(seq3d, delta[None, :], W[:, 0][None, :], b[None, :])
    return out_t.T


# speedup vs baseline: 1.1280x; 1.1280x over previous
import jax
import jax.numpy as jnp
from jax import lax
from jax.experimental import pallas as pl
from jax.experimental.pallas import tpu as pltpu

_SEQ = 2048
_NOUT = 1000
_NIN = 1000
_TBLK = 1024
_SUBL = _TBLK // 128


def _body(seq_ref, delta_ref, w_ref, b_ref, o_ref, srow, wbc, dscr, carry, wstat):
    i = pl.program_id(0)

    @pl.when(i == 0)
    def _():
        carry[0, 0] = jnp.float32(0.0)
        r0 = lax.broadcasted_iota(jnp.int32, (_NOUT, _NOUT), 0)
        c0 = lax.broadcasted_iota(jnp.int32, (_NOUT, _NOUT), 1)
        iden = (r0 == c0).astype(jnp.float32)
        tdims = (((1,), (1,)), ((), ()))
        wbc[:, 0:1] = lax.dot_general(
            iden, w_ref[...], tdims,
            preferred_element_type=jnp.float32,
            precision=lax.Precision.HIGHEST,
        )
        wbc[:, 1:2] = lax.dot_general(
            iden, b_ref[...], tdims,
            preferred_element_type=jnp.float32,
            precision=lax.Precision.HIGHEST,
        )
        dscr[0:1, 0:_NIN] = delta_ref[...]
        wstat[0, 0] = jnp.max(w_ref[...])
        wstat[0, 1] = jnp.min(w_ref[...])
        wstat[0, 2] = jnp.max(b_ref[...])

    for k in range(_SUBL):
        srow[0:1, k * 128 : (k + 1) * 128] = seq_ref[0, k : k + 1, :]

    dnums = lax.GatherDimensionNumbers(
        offset_dims=(),
        collapsed_slice_dims=(1,),
        start_index_map=(1,),
        operand_batching_dims=(0,),
        start_indices_batching_dims=(0,),
    )
    idx = srow[...]
    idxm = (idx & 127)[:, :, None]
    idxh = idx >> 7
    g = jnp.zeros((1, _TBLK), jnp.float32)
    for c in range(1024 // 128):
        gc = lax.gather(
            dscr[:, c * 128 : (c + 1) * 128],
            idxm,
            dimension_numbers=dnums,
            slice_sizes=(1, 1),
            mode=lax.GatherScatterMode.PROMISE_IN_BOUNDS,
        )
        g = jnp.where(idxh == c, gc, g)

    lane = lax.broadcasted_iota(jnp.int32, (1, _TBLK), 1)
    csum = g
    s = 1
    while s < _TBLK:
        rolled = pltpu.roll(csum, s, 1)
        csum = csum + jnp.where(lane >= s, rolled, jnp.float32(0.0))
        s *= 2
    c0 = carry[0, 0]
    counters = csum + c0
    carry[0, 0] = c0 + jnp.sum(g)

    mb = jnp.maximum(counters * wstat[0, 0], counters * wstat[0, 1]) + wstat[0, 2]
    e = jnp.exp(wbc[:, 0:1] * counters + (wbc[:, 1:2] - mb))
    s = lax.dot_general(
        jnp.ones((1, _NOUT), jnp.float32), e, (((1,), (0,)), ((), ())),
        preferred_element_type=jnp.float32,
    )
    o_ref[...] = e * (1.0 / s)


def kernel(input_seq, delta, W, b):
    seq3d = input_seq.reshape(_SEQ // _TBLK, _SUBL, 128)
    out_t = pl.pallas_call(
        _body,
        grid=(_SEQ // _TBLK,),
        in_specs=[
            pl.BlockSpec((1, _SUBL, 128), lambda i: (i, 0, 0)),
            pl.BlockSpec((1, _NIN), lambda i: (0, 0)),
            pl.BlockSpec((1, _NOUT), lambda i: (0, 0)),
            pl.BlockSpec((1, _NOUT), lambda i: (0, 0)),
        ],
        out_specs=pl.BlockSpec((_NOUT, _TBLK), lambda i: (0, i)),
        out_shape=jax.ShapeDtypeStruct((_NOUT, _SEQ), jnp.float32),
        scratch_shapes=[
            pltpu.VMEM((1, _TBLK), jnp.int32),
            pltpu.VMEM((_NOUT, 2), jnp.float32),
            pltpu.VMEM((1, 1024), jnp.float32),
            pltpu.SMEM((1, 1), jnp.float32),
            pltpu.SMEM((1, 3), jnp.float32),
        ],
    )(seq3d, delta[None, :], W[:, 0][None, :], b[None, :])
    return out_t.T


# speedup vs baseline: 1.2399x; 1.0992x over previous
import jax
import jax.numpy as jnp
from jax import lax
from jax.experimental import pallas as pl
from jax.experimental.pallas import tpu as pltpu

_SEQ = 2048
_NOUT = 1000
_NIN = 1000
_TBLK = 1024
_LOG2E = 1.4426950408889634


def _body(seq_ref, delta_ref, w_ref, b_ref, o_ref, srow, cnt, wbc, dscr, wstat):
    i = pl.program_id(0)

    @pl.when(i == 0)
    def _():
        r0 = lax.broadcasted_iota(jnp.int32, (_NOUT, _NOUT), 0)
        c0 = lax.broadcasted_iota(jnp.int32, (_NOUT, _NOUT), 1)
        iden = (r0 == c0).astype(jnp.float32)
        tdims = (((1,), (1,)), ((), ()))
        w2 = w_ref[...] * _LOG2E
        b2 = b_ref[...] * _LOG2E
        wbc[:, 0:1] = lax.dot_general(
            iden, w2, tdims,
            preferred_element_type=jnp.float32,
            precision=lax.Precision.HIGHEST,
        )
        wbc[:, 1:2] = lax.dot_general(
            iden, b2, tdims,
            preferred_element_type=jnp.float32,
            precision=lax.Precision.HIGHEST,
        )
        wstat[0, 0] = jnp.max(w2)
        wstat[0, 1] = jnp.min(w2)
        wstat[0, 2] = jnp.max(b2)
        dscr[0:1, 0:_NIN] = delta_ref[...]

        for k in range(_SEQ // 128):
            srow[0:1, k * 128 : (k + 1) * 128] = seq_ref[0, k : k + 1, :]

        dnums = lax.GatherDimensionNumbers(
            offset_dims=(),
            collapsed_slice_dims=(1,),
            start_index_map=(1,),
            operand_batching_dims=(0,),
            start_indices_batching_dims=(0,),
        )
        idx = srow[...]
        idxm = (idx & 127)[:, :, None]
        idxh = idx >> 7
        g = jnp.zeros((1, _SEQ), jnp.float32)
        for c in range(1024 // 128):
            gc = lax.gather(
                dscr[:, c * 128 : (c + 1) * 128],
                idxm,
                dimension_numbers=dnums,
                slice_sizes=(1, 1),
                mode=lax.GatherScatterMode.PROMISE_IN_BOUNDS,
            )
            g = jnp.where(idxh == c, gc, g)

        lane = lax.broadcasted_iota(jnp.int32, (1, _SEQ), 1)
        csum = g
        s = 1
        while s < _SEQ:
            rolled = pltpu.roll(csum, s, 1)
            csum = csum + jnp.where(lane >= s, rolled, jnp.float32(0.0))
            s *= 2
        cnt[...] = csum

    counters = cnt[0:1, pl.ds(i * _TBLK, _TBLK)]

    mb = jnp.maximum(counters * wstat[0, 0], counters * wstat[0, 1]) + wstat[0, 2]
    e = jnp.exp2(wbc[:, 0:1] * counters + (wbc[:, 1:2] - mb))
    s = lax.dot_general(
        jnp.ones((1, _NOUT), jnp.float32), e, (((1,), (0,)), ((), ())),
        preferred_element_type=jnp.float32,
    )
    o_ref[...] = e * (1.0 / s)


def kernel(input_seq, delta, W, b):
    seq3d = input_seq.reshape(1, _SEQ // 128, 128)
    out_t = pl.pallas_call(
        _body,
        grid=(_SEQ // _TBLK,),
        in_specs=[
            pl.BlockSpec((1, _SEQ // 128, 128), lambda i: (0, 0, 0)),
            pl.BlockSpec((1, _NIN), lambda i: (0, 0)),
            pl.BlockSpec((1, _NOUT), lambda i: (0, 0)),
            pl.BlockSpec((1, _NOUT), lambda i: (0, 0)),
        ],
        out_specs=pl.BlockSpec((_NOUT, _TBLK), lambda i: (0, i)),
        out_shape=jax.ShapeDtypeStruct((_NOUT, _SEQ), jnp.float32),
        scratch_shapes=[
            pltpu.VMEM((1, _SEQ), jnp.int32),
            pltpu.VMEM((1, _SEQ), jnp.float32),
            pltpu.VMEM((_NOUT, 2), jnp.float32),
            pltpu.VMEM((1, 1024), jnp.float32),
            pltpu.SMEM((1, 3), jnp.float32),
        ],
    )(seq3d, delta[None, :], W[:, 0][None, :], b[None, :])
    return out_t.T
